# Initial kernel scaffold; baseline (speedup 1.0000x reference)
import jax
import jax.numpy as jnp
from jax.experimental import pallas as pl


def kernel(x_visit, x_entity, edge_index_vv, edge_index_ve, edge_index_ev, edge_index_vrev, params):
    raise NotImplementedError("write your pallas kernel here")


# speedup vs baseline: 1.7595x; 1.7595x over previous
import jax
import jax.numpy as jnp
from jax.experimental import pallas as pl
from jax.experimental.pallas import tpu as pltpu


def _mm_body(x_ref, wt_ref, b_ref, o_ref):
    o_ref[...] = (
        jnp.dot(x_ref[...], wt_ref[...], preferred_element_type=jnp.float32)
        + b_ref[...]
    )


def _proj(x, Wt, b, bm=1000):
    N, K = x.shape
    Dout = Wt.shape[1]
    return pl.pallas_call(
        _mm_body,
        grid=(N // bm,),
        in_specs=[
            pl.BlockSpec((bm, K), lambda i: (i, 0)),
            pl.BlockSpec((K, Dout), lambda i: (0, 0)),
            pl.BlockSpec((1, Dout), lambda i: (0, 0)),
        ],
        out_specs=pl.BlockSpec((bm, Dout), lambda i: (i, 0)),
        out_shape=jax.ShapeDtypeStruct((N, Dout), jnp.float32),
    )(x, Wt, b.reshape(1, -1))


def _gat(x_src, x_dst, edge_index, cp, num_dst):
    h = x_src @ cp['W'].T
    asn = x_src @ (cp['W'].T @ cp['a_src'])
    adn = x_dst @ (cp['W'].T @ cp['a_dst'])
    src, dst = edge_index[0], edge_index[1]
    e = asn[src] + adn[dst]
    e = jnp.where(e >= 0, e, 0.2 * e)
    ex = jnp.exp(e)
    den = jax.ops.segment_sum(ex, dst, num_segments=num_dst)
    num = jax.ops.segment_sum(ex[:, None] * h[src], dst, num_segments=num_dst)
    return num / (den[:, None] + 1e-16) + cp['b']


def kernel(x_visit, x_entity, edge_index_vv, edge_index_ve, edge_index_ev,
           edge_index_vrev, params):
    del edge_index_vrev
    p = params
    hv = _proj(x_visit, p['W_vin'].T, p['b_vin'])
    he = _proj(x_entity, p['W_ein'].T, p['b_ein'])
    nv = hv.shape[0]
    ne = he.shape[0]
    ov = (_gat(hv, hv, edge_index_vv, p['c1_vv'], nv)
          + _gat(he, hv, edge_index_ev, p['c1_ev'], nv))
    oe = _gat(hv, he, edge_index_ve, p['c1_ve'], ne)
    hv = jax.nn.relu(ov)
    he = jax.nn.relu(oe)
    ov = (_gat(hv, hv, edge_index_vv, p['c2_vv'], nv)
          + _gat(he, hv, edge_index_ev, p['c2_ev'], nv))
    oe = _gat(hv, he, edge_index_ve, p['c2_ve'], ne)
    hv = jax.nn.relu(ov)
    logits = _proj(hv, p['W_out'].T, p['b_out'])
    return logits


# speedup vs baseline: 2.4154x; 1.3728x over previous
import functools

import jax
import jax.numpy as jnp
from jax import lax
from jax.experimental import pallas as pl
from jax.experimental.pallas import tpu as pltpu
from jax.experimental.pallas import tpu_sc as plsc

_E = 800000
_N = 50000
_EPT = 50176
_EPAD = 16 * _EPT
_IDXROWS = _EPT // 128
_G = 4
_STEPS = _IDXROWS // _G
_NPAD = 12512
_NHALF = 12500


def _mm_body(x_ref, wt_ref, b_ref, o_ref):
    o_ref[...] = (
        jnp.dot(x_ref[...], wt_ref[...], preferred_element_type=jnp.float32)
        + b_ref[...]
    )


def _proj(x, Wt, b, bm=1000):
    N, K = x.shape
    Dout = Wt.shape[1]
    return pl.pallas_call(
        _mm_body,
        grid=(N // bm,),
        in_specs=[
            pl.BlockSpec((bm, K), lambda i: (i, 0)),
            pl.BlockSpec((K, Dout), lambda i: (0, 0)),
            pl.BlockSpec((1, Dout), lambda i: (0, 0)),
        ],
        out_specs=pl.BlockSpec((bm, Dout), lambda i: (i, 0)),
        out_shape=jax.ShapeDtypeStruct((N, Dout), jnp.float32),
    )(x, Wt, b.reshape(1, -1))


def _comb2_body(a0, a1, a2, a3, da, b0, b1, b2, b3, db, ba, bb, o_ref):
    na = jnp.concatenate([a0[...], a1[...], a2[...], a3[...]], axis=1)
    nb = jnp.concatenate([b0[...], b1[...], b2[...], b3[...]], axis=1)
    den_a = da[...][:, 0:1] + 1e-16
    den_b = db[...][:, 0:1] + 1e-16
    o_ref[...] = jnp.maximum(na / den_a + ba[...] + nb / den_b + bb[...], 0.0)


def _comb2(qa, da, qb, db, ba, bb, bm=1000):
    N = qa[0].shape[0]
    q_spec = pl.BlockSpec((bm, 16), lambda i: (i, 0))
    b_spec = pl.BlockSpec((1, 64), lambda i: (0, 0))
    return pl.pallas_call(
        _comb2_body,
        grid=(N // bm,),
        in_specs=[q_spec] * 5 + [q_spec] * 5 + [b_spec, b_spec],
        out_specs=pl.BlockSpec((bm, 64), lambda i: (i, 0)),
        out_shape=jax.ShapeDtypeStruct((N, 64), jnp.float32),
    )(*qa, da, *qb, db, ba.reshape(1, -1), bb.reshape(1, -1))


def _comb1_body(a0, a1, a2, a3, da, ba, o_ref):
    na = jnp.concatenate([a0[...], a1[...], a2[...], a3[...]], axis=1)
    den_a = da[...][:, 0:1] + 1e-16
    o_ref[...] = jnp.maximum(na / den_a + ba[...], 0.0)


def _comb1(qa, da, ba, bm=1000):
    N = qa[0].shape[0]
    q_spec = pl.BlockSpec((bm, 16), lambda i: (i, 0))
    b_spec = pl.BlockSpec((1, 64), lambda i: (0, 0))
    return pl.pallas_call(
        _comb1_body,
        grid=(N // bm,),
        in_specs=[q_spec] * 5 + [b_spec],
        out_specs=pl.BlockSpec((bm, 64), lambda i: (i, 0)),
        out_shape=jax.ShapeDtypeStruct((N, 64), jnp.float32),
    )(*qa, da, ba.reshape(1, -1))


def _sc_edge_kernel(src_hbm, dst_hbm, asn_hbm, adn_hbm, nb_hbm,
                    h0_hbm, h1_hbm, num_hbm,
                    asn_v, adn_v, nb_v, src_v, dst_v, dstl_v, ex_v, rows_v,
                    acc_sh, sem):
    c = lax.axis_index("c")
    s = lax.axis_index("s")
    zeros16 = jnp.zeros((16,), jnp.float32)

    def zrow(i, _):
        for g in range(_G):
            rows_v[g, i, 0:16] = zeros16
        return 0
    lax.fori_loop(0, 128, zrow, 0)

    r0 = s * 784

    @pl.when(s < 15)
    def _():
        def zc(i, _):
            pltpu.sync_copy(rows_v.at[0], acc_sh.at[pl.ds(r0 + i * 128, 128)])
            return 0
        lax.fori_loop(0, 6, zc, 0)
        pltpu.sync_copy(rows_v.at[0].at[pl.ds(0, 16)],
                        acc_sh.at[pl.ds(r0 + 768, 16)])

    @pl.when(s == 15)
    def _():
        def zc(i, _):
            pltpu.sync_copy(rows_v.at[0], acc_sh.at[pl.ds(r0 + i * 128, 128)])
            return 0
        lax.fori_loop(0, 5, zc, 0)
        pltpu.sync_copy(rows_v.at[0].at[pl.ds(0, 112)],
                        acc_sh.at[pl.ds(r0 + 640, 112)])

    pltpu.sync_copy(asn_hbm, asn_v)
    pltpu.sync_copy(adn_hbm, adn_v)
    pltpu.sync_copy(nb_hbm, nb_v)
    plsc.subcore_barrier()

    rowbase = s * _IDXROWS

    def step(t, _):
        rb = rowbase + t * _G
        pltpu.sync_copy(src_hbm.at[pl.ds(rb, _G)], src_v)
        pltpu.sync_copy(dst_hbm.at[pl.ds(rb, _G)], dst_v)
        for g in range(_G):
            @pl.when(c == 0)
            def _(g=g):
                pltpu.async_copy(h0_hbm.at[src_v.at[g]], rows_v.at[g],
                                 sem).wait()

            @pl.when(c == 1)
            def _(g=g):
                pltpu.async_copy(h1_hbm.at[src_v.at[g]], rows_v.at[g],
                                 sem).wait()
        nbv = nb_v[0:16]
        for g in range(_G):
            def grp(j, _):
                sl = pl.ds(j * 16, 16)
                sv = src_v[g, sl]
                dv = dst_v[g, sl]
                a = plsc.load_gather(asn_v, [sv])
                b = plsc.load_gather(adn_v, [dv])
                e = a + b
                e = jnp.where(e >= 0.0, e, e * 0.2)
                ex_v[g, sl] = jnp.exp(e)
                dl = dv - nbv
                dl = jnp.where((dl >= 0) & (dl < _NHALF), dl, _NHALF)
                dstl_v[g, sl] = dl
                return 0
            lax.fori_loop(0, 8, grp, 0)
        for g in range(_G):
            def rw(j, _):
                exg = ex_v[g, pl.ds(j * 16, 16)]
                base = j * 16
                for l in range(16):
                    exs = exg[l]
                    k2 = base + l
                    rows_v[g, k2, 0:16] = rows_v[g, k2, 0:16] * exs
                return 0
            lax.fori_loop(0, 8, rw, 0)
        for g in range(_G):
            pltpu.sync_copy(rows_v.at[g], acc_sh.at[dstl_v.at[g]], add=True)
        return 0

    lax.fori_loop(0, _STEPS, step, 0)
    plsc.subcore_barrier()

    @pl.when(s < 15)
    def _():
        def wc(i, _):
            sl = pl.ds(r0 + i * 128, 128)
            pltpu.sync_copy(acc_sh.at[sl], num_hbm.at[c].at[sl])
            return 0
        lax.fori_loop(0, 6, wc, 0)
        sl16 = pl.ds(r0 + 768, 16)
        pltpu.sync_copy(acc_sh.at[sl16], num_hbm.at[c].at[sl16])

    @pl.when(s == 15)
    def _():
        def wc(i, _):
            sl = pl.ds(r0 + i * 128, 128)
            pltpu.sync_copy(acc_sh.at[sl], num_hbm.at[c].at[sl])
            return 0
        lax.fori_loop(0, 5, wc, 0)
        sl112 = pl.ds(r0 + 640, 112)
        pltpu.sync_copy(acc_sh.at[sl112], num_hbm.at[c].at[sl112])


_sc_edge = functools.partial(
    pl.kernel,
    mesh=plsc.VectorSubcoreMesh(core_axis_name="c", subcore_axis_name="s"),
    compiler_params=pltpu.CompilerParams(
        needs_layout_passes=False, use_tc_tiling_on_sc=False),
    out_type=jax.ShapeDtypeStruct((2, _NPAD, 16), jnp.float32),
    scratch_types=[
        pltpu.VMEM((_N,), jnp.float32),
        pltpu.VMEM((_N,), jnp.float32),
        pltpu.VMEM((16,), jnp.int32),
        pltpu.VMEM((_G, 128), jnp.int32),
        pltpu.VMEM((_G, 128), jnp.int32),
        pltpu.VMEM((_G, 128), jnp.int32),
        pltpu.VMEM((_G, 128), jnp.float32),
        pltpu.VMEM((_G, 128, 16), jnp.float32),
        pltpu.VMEM_SHARED((_NPAD, 16), jnp.float32),
        pltpu.SemaphoreType.DMA,
    ],
)(_sc_edge_kernel)


def _prep_edges(ei):
    src = jnp.pad(ei[0], (0, _EPAD - _E))
    dst = jnp.pad(ei[1], (0, _EPAD - _E), constant_values=_N)
    return src.reshape(-1, 128), dst.reshape(-1, 128)


def _gat_prep(x_src, x_dst, cp):
    ws = cp['W'].T @ cp['a_src']
    wd = cp['W'].T @ cp['a_dst']
    ha = _proj(x_src, jnp.concatenate([cp['W'].T, ws[:, None]], axis=1),
               jnp.zeros((65,), jnp.float32))
    asn = ha[:, 64]
    adn = _proj(x_dst, wd[:, None], jnp.zeros((1,), jnp.float32))[:, 0]
    ones = jnp.ones((ha.shape[0], 16), jnp.float32)
    hh = jnp.stack([
        jnp.stack([ha[:, 0:16], ha[:, 16:32]]),
        jnp.stack([ha[:, 32:48], ha[:, 48:64]]),
        jnp.stack([ones, ones]),
    ])
    return hh, asn, adn


def _edge_phase(src2s, dst2s, hhs, asns, adns, tok):
    nbs = jnp.stack([jnp.full((16,), q * _NHALF, jnp.int32)
                     for q in range(4)])

    def inner(tok2, it):
        hpair = lax.dynamic_index_in_dim(inner.hh, it // 4, 0, False)
        nb = lax.dynamic_index_in_dim(nbs, it % 4, 0, False)
        asn_g, _ = lax.optimization_barrier((inner.asn, tok2))
        num = _sc_edge(inner.src2, inner.dst2, asn_g, inner.adn, nb,
                       hpair[0], hpair[1])
        return num[0, 0, 0], num

    def body(tok2, xs):
        src2, dst2, hh, asn, adn = xs
        inner.src2, inner.dst2, inner.hh = src2, dst2, hh
        inner.asn, inner.adn = asn, adn
        return lax.scan(inner, tok2, jnp.arange(12, dtype=jnp.int32))

    tok, nums = lax.scan(body, tok, (src2s, dst2s, hhs, asns, adns))
    return nums, tok


def _quarters(nums, g):
    def full(it, c2):
        return jnp.concatenate(
            [nums[g, 4 * it + q, c2, :_NHALF] for q in range(4)], axis=0)
    return ([full(0, 0), full(0, 1), full(1, 0), full(1, 1)], full(2, 0))


def kernel(x_visit, x_entity, edge_index_vv, edge_index_ve, edge_index_ev,
           edge_index_vrev, params):
    del edge_index_vrev
    p = params
    vv = _prep_edges(edge_index_vv)
    ve = _prep_edges(edge_index_ve)
    ev = _prep_edges(edge_index_ev)

    hv = _proj(x_visit, p['W_vin'].T, p['b_vin'])
    he = _proj(x_entity, p['W_ein'].T, p['b_ein'])

    tok = jnp.float32(0.0)

    h_a, asn_a, adn_a = _gat_prep(hv, hv, p['c1_vv'])
    h_b, asn_b, adn_b = _gat_prep(he, hv, p['c1_ev'])
    h_c, asn_c, adn_c = _gat_prep(hv, he, p['c1_ve'])
    nums1, tok = _edge_phase(
        jnp.stack([vv[0], ev[0], ve[0]]), jnp.stack([vv[1], ev[1], ve[1]]),
        jnp.stack([h_a, h_b, h_c]), jnp.stack([asn_a, asn_b, asn_c]),
        jnp.stack([adn_a, adn_b, adn_c]), tok)
    qa, da = _quarters(nums1, 0)
    qb, db = _quarters(nums1, 1)
    qc, dc = _quarters(nums1, 2)
    hv1 = _comb2(qa, da, qb, db, p['c1_vv']['b'], p['c1_ev']['b'])
    he1 = _comb1(qc, dc, p['c1_ve']['b'])

    h_a, asn_a, adn_a = _gat_prep(hv1, hv1, p['c2_vv'])
    h_b, asn_b, adn_b = _gat_prep(he1, hv1, p['c2_ev'])
    nums2, tok = _edge_phase(
        jnp.stack([vv[0], ev[0]]), jnp.stack([vv[1], ev[1]]),
        jnp.stack([h_a, h_b]), jnp.stack([asn_a, asn_b]),
        jnp.stack([adn_a, adn_b]), tok)
    qa, da = _quarters(nums2, 0)
    qb, db = _quarters(nums2, 1)
    hv2 = _comb2(qa, da, qb, db, p['c2_vv']['b'], p['c2_ev']['b'])

    logits = _proj(hv2, p['W_out'].T, p['b_out'])
    return logits


# speedup vs baseline: 3.4019x; 1.4084x over previous
import functools

import jax
import jax.numpy as jnp
from jax import lax
from jax.experimental import pallas as pl
from jax.experimental.pallas import tpu as pltpu
from jax.experimental.pallas import tpu_sc as plsc

_E = 800000
_N = 50000
_EPT = 50176
_EPAD = 16 * _EPT
_IDXROWS = _EPT // 128
_G = 4
_STEPS = _IDXROWS // _G
_NPAD = 16672
_NHALF = 16667


def _mm_body(x_ref, wt_ref, b_ref, o_ref):
    o_ref[...] = (
        jnp.dot(x_ref[...], wt_ref[...], preferred_element_type=jnp.float32)
        + b_ref[...]
    )


def _proj(x, Wt, b, bm=1000):
    N, K = x.shape
    Dout = Wt.shape[1]
    return pl.pallas_call(
        _mm_body,
        grid=(N // bm,),
        in_specs=[
            pl.BlockSpec((bm, K), lambda i: (i, 0)),
            pl.BlockSpec((K, Dout), lambda i: (0, 0)),
            pl.BlockSpec((1, Dout), lambda i: (0, 0)),
        ],
        out_specs=pl.BlockSpec((bm, Dout), lambda i: (i, 0)),
        out_shape=jax.ShapeDtypeStruct((N, Dout), jnp.float32),
    )(x, Wt, b.reshape(1, -1))


def _comb2_body(a0, a1, a2, a3, da, b0, b1, b2, b3, db, ba, bb, o_ref):
    na = jnp.concatenate([a0[...], a1[...], a2[...], a3[...]], axis=1)
    nb = jnp.concatenate([b0[...], b1[...], b2[...], b3[...]], axis=1)
    den_a = da[...][:, 0:1] + 1e-16
    den_b = db[...][:, 0:1] + 1e-16
    o_ref[...] = jnp.maximum(na / den_a + ba[...] + nb / den_b + bb[...], 0.0)


def _comb2(qa, da, qb, db, ba, bb, bm=1000):
    N = qa[0].shape[0]
    q_spec = pl.BlockSpec((bm, 16), lambda i: (i, 0))
    b_spec = pl.BlockSpec((1, 64), lambda i: (0, 0))
    return pl.pallas_call(
        _comb2_body,
        grid=(N // bm,),
        in_specs=[q_spec] * 5 + [q_spec] * 5 + [b_spec, b_spec],
        out_specs=pl.BlockSpec((bm, 64), lambda i: (i, 0)),
        out_shape=jax.ShapeDtypeStruct((N, 64), jnp.float32),
    )(*qa, da, *qb, db, ba.reshape(1, -1), bb.reshape(1, -1))


def _comb1_body(a0, a1, a2, a3, da, ba, o_ref):
    na = jnp.concatenate([a0[...], a1[...], a2[...], a3[...]], axis=1)
    den_a = da[...][:, 0:1] + 1e-16
    o_ref[...] = jnp.maximum(na / den_a + ba[...], 0.0)


def _comb1(qa, da, ba, bm=1000):
    N = qa[0].shape[0]
    q_spec = pl.BlockSpec((bm, 16), lambda i: (i, 0))
    b_spec = pl.BlockSpec((1, 64), lambda i: (0, 0))
    return pl.pallas_call(
        _comb1_body,
        grid=(N // bm,),
        in_specs=[q_spec] * 5 + [b_spec],
        out_specs=pl.BlockSpec((bm, 64), lambda i: (i, 0)),
        out_shape=jax.ShapeDtypeStruct((N, 64), jnp.float32),
    )(*qa, da, ba.reshape(1, -1))


def _sc_edge_kernel(src_hbm, dst_hbm, asn_hbm, adn_hbm, nb_hbm,
                    h0_hbm, h1_hbm, num_hbm,
                    asn_v, adn_v, nb_v, src_v, dst_v, dstl_v, ex_v, rows_v,
                    acc_sh, sem):
    c = lax.axis_index("c")
    s = lax.axis_index("s")
    zeros16 = jnp.zeros((16,), jnp.float32)

    def zrow(i, _):
        for g in range(_G):
            rows_v[g, i, 0:16] = zeros16
        return 0
    lax.fori_loop(0, 128, zrow, 0)

    r0 = s * 1048

    @pl.when(s < 15)
    def _():
        def zc(i, _):
            pltpu.sync_copy(rows_v.at[0], acc_sh.at[pl.ds(r0 + i * 128, 128)])
            return 0
        lax.fori_loop(0, 8, zc, 0)
        pltpu.sync_copy(rows_v.at[0].at[pl.ds(0, 24)],
                        acc_sh.at[pl.ds(r0 + 1024, 24)])

    @pl.when(s == 15)
    def _():
        def zc(i, _):
            pltpu.sync_copy(rows_v.at[0], acc_sh.at[pl.ds(r0 + i * 128, 128)])
            return 0
        lax.fori_loop(0, 7, zc, 0)
        pltpu.sync_copy(rows_v.at[0].at[pl.ds(0, 56)],
                        acc_sh.at[pl.ds(r0 + 896, 56)])

    pltpu.sync_copy(asn_hbm, asn_v)
    pltpu.sync_copy(adn_hbm, adn_v)
    pltpu.sync_copy(nb_hbm, nb_v)
    plsc.subcore_barrier()

    rowbase = s * _IDXROWS

    def step(t, _):
        rb = rowbase + t * _G
        pltpu.sync_copy(src_hbm.at[pl.ds(rb, _G)], src_v)
        pltpu.sync_copy(dst_hbm.at[pl.ds(rb, _G)], dst_v)
        for g in range(_G):
            @pl.when(c == 0)
            def _(g=g):
                pltpu.async_copy(h0_hbm.at[src_v.at[g]], rows_v.at[g],
                                 sem).wait()

            @pl.when(c == 1)
            def _(g=g):
                pltpu.async_copy(h1_hbm.at[src_v.at[g]], rows_v.at[g],
                                 sem).wait()
        nbv = nb_v[0:16]
        for g in range(_G):
            def grp(j, _):
                sl = pl.ds(j * 16, 16)
                sv = src_v[g, sl]
                dv = dst_v[g, sl]
                a = plsc.load_gather(asn_v, [sv])
                b = plsc.load_gather(adn_v, [dv])
                e = a + b
                e = jnp.where(e >= 0.0, e, e * 0.2)
                ex_v[g, sl] = jnp.exp(e)
                dl = dv - nbv
                dl = jnp.where((dl >= 0) & (dl < _NHALF), dl, _NHALF)
                dstl_v[g, sl] = dl
                return 0
            lax.fori_loop(0, 8, grp, 0)
        for g in range(_G):
            def rw(j, _):
                exg = ex_v[g, pl.ds(j * 16, 16)]
                base = j * 16
                for l in range(16):
                    exs = exg[l]
                    k2 = base + l
                    rows_v[g, k2, 0:16] = rows_v[g, k2, 0:16] * exs
                return 0
            lax.fori_loop(0, 8, rw, 0)
        for g in range(_G):
            pltpu.sync_copy(rows_v.at[g], acc_sh.at[dstl_v.at[g]], add=True)
        return 0

    lax.fori_loop(0, _STEPS, step, 0)
    plsc.subcore_barrier()

    @pl.when(s < 15)
    def _():
        def wc(i, _):
            sl = pl.ds(r0 + i * 128, 128)
            pltpu.sync_copy(acc_sh.at[sl], num_hbm.at[c].at[sl])
            return 0
        lax.fori_loop(0, 8, wc, 0)
        sl24 = pl.ds(r0 + 1024, 24)
        pltpu.sync_copy(acc_sh.at[sl24], num_hbm.at[c].at[sl24])

    @pl.when(s == 15)
    def _():
        def wc(i, _):
            sl = pl.ds(r0 + i * 128, 128)
            pltpu.sync_copy(acc_sh.at[sl], num_hbm.at[c].at[sl])
            return 0
        lax.fori_loop(0, 7, wc, 0)
        sl56 = pl.ds(r0 + 896, 56)
        pltpu.sync_copy(acc_sh.at[sl56], num_hbm.at[c].at[sl56])


_sc_edge = functools.partial(
    pl.kernel,
    mesh=plsc.VectorSubcoreMesh(core_axis_name="c", subcore_axis_name="s"),
    compiler_params=pltpu.CompilerParams(
        needs_layout_passes=False, use_tc_tiling_on_sc=False),
    out_type=jax.ShapeDtypeStruct((2, _NPAD, 16), jnp.float32),
    scratch_types=[
        pltpu.VMEM((_N,), jnp.float32),
        pltpu.VMEM((_N,), jnp.float32),
        pltpu.VMEM((16,), jnp.int32),
        pltpu.VMEM((_G, 128), jnp.int32),
        pltpu.VMEM((_G, 128), jnp.int32),
        pltpu.VMEM((_G, 128), jnp.int32),
        pltpu.VMEM((_G, 128), jnp.float32),
        pltpu.VMEM((_G, 128, 16), jnp.float32),
        pltpu.VMEM_SHARED((_NPAD, 16), jnp.float32),
        pltpu.SemaphoreType.DMA,
    ],
)(_sc_edge_kernel)


def _prep_edges(ei):
    src = jnp.pad(ei[0], (0, _EPAD - _E))
    dst = jnp.pad(ei[1], (0, _EPAD - _E), constant_values=_N)
    return src.reshape(-1, 128), dst.reshape(-1, 128)


def _gat_prep(x_src, x_dst, cp):
    ws = cp['W'].T @ cp['a_src']
    wd = cp['W'].T @ cp['a_dst']
    ha = _proj(x_src, jnp.concatenate([cp['W'].T, ws[:, None]], axis=1),
               jnp.zeros((65,), jnp.float32))
    asn = ha[:, 64]
    adn = _proj(x_dst, wd[:, None], jnp.zeros((1,), jnp.float32))[:, 0]
    ones = jnp.ones((ha.shape[0], 16), jnp.float32)
    hh = jnp.stack([
        jnp.stack([ha[:, 0:16], ha[:, 16:32]]),
        jnp.stack([ha[:, 32:48], ha[:, 48:64]]),
        jnp.stack([ones, ones]),
    ])
    return hh, asn, adn


def _edge_phase(src2s, dst2s, hhs, asns, adns, tok):
    nbs = jnp.stack([jnp.full((16,), q * _NHALF, jnp.int32)
                     for q in range(3)])

    def inner(tok2, it):
        hpair = lax.dynamic_index_in_dim(inner.hh, it // 3, 0, False)
        nb = lax.dynamic_index_in_dim(nbs, it % 3, 0, False)
        asn_g, _ = lax.optimization_barrier((inner.asn, tok2))
        num = _sc_edge(inner.src2, inner.dst2, asn_g, inner.adn, nb,
                       hpair[0], hpair[1])
        return num[0, 0, 0], num

    def body(tok2, xs):
        src2, dst2, hh, asn, adn = xs
        inner.src2, inner.dst2, inner.hh = src2, dst2, hh
        inner.asn, inner.adn = asn, adn
        return lax.scan(inner, tok2, jnp.arange(9, dtype=jnp.int32))

    tok, nums = lax.scan(body, tok, (src2s, dst2s, hhs, asns, adns))
    return nums, tok


def _quarters(nums, g):
    def full(it, c2):
        parts = [nums[g, 3 * it + q, c2, :_NHALF] for q in range(3)]
        return jnp.concatenate(parts, axis=0)[:_N]
    return ([full(0, 0), full(0, 1), full(1, 0), full(1, 1)], full(2, 0))


def kernel(x_visit, x_entity, edge_index_vv, edge_index_ve, edge_index_ev,
           edge_index_vrev, params):
    del edge_index_vrev
    p = params
    vv = _prep_edges(edge_index_vv)
    ve = _prep_edges(edge_index_ve)
    ev = _prep_edges(edge_index_ev)

    hv = _proj(x_visit, p['W_vin'].T, p['b_vin'])
    he = _proj(x_entity, p['W_ein'].T, p['b_ein'])

    tok = jnp.float32(0.0)

    h_a, asn_a, adn_a = _gat_prep(hv, hv, p['c1_vv'])
    h_b, asn_b, adn_b = _gat_prep(he, hv, p['c1_ev'])
    h_c, asn_c, adn_c = _gat_prep(hv, he, p['c1_ve'])
    nums1, tok = _edge_phase(
        jnp.stack([vv[0], ev[0], ve[0]]), jnp.stack([vv[1], ev[1], ve[1]]),
        jnp.stack([h_a, h_b, h_c]), jnp.stack([asn_a, asn_b, asn_c]),
        jnp.stack([adn_a, adn_b, adn_c]), tok)
    qa, da = _quarters(nums1, 0)
    qb, db = _quarters(nums1, 1)
    qc, dc = _quarters(nums1, 2)
    hv1 = _comb2(qa, da, qb, db, p['c1_vv']['b'], p['c1_ev']['b'])
    he1 = _comb1(qc, dc, p['c1_ve']['b'])

    h_a, asn_a, adn_a = _gat_prep(hv1, hv1, p['c2_vv'])
    h_b, asn_b, adn_b = _gat_prep(he1, hv1, p['c2_ev'])
    nums2, tok = _edge_phase(
        jnp.stack([vv[0], ev[0]]), jnp.stack([vv[1], ev[1]]),
        jnp.stack([h_a, h_b]), jnp.stack([asn_a, asn_b]),
        jnp.stack([adn_a, adn_b]), tok)
    qa, da = _quarters(nums2, 0)
    qb, db = _quarters(nums2, 1)
    hv2 = _comb2(qa, da, qb, db, p['c2_vv']['b'], p['c2_ev']['b'])

    logits = _proj(hv2, p['W_out'].T, p['b_out'])
    return logits
